# baseline (speedup 1.0000x reference)
import functools

import jax
import jax.numpy as jnp
from jax.experimental import pallas as pl
from jax.experimental.pallas import tpu as pltpu

_ALPHA = 0.2


def _leaky(x):
    return jnp.where(x > 0, x, _ALPHA * x)


def _pick_bm(m, candidates):
    for c in candidates:
        if m % c == 0:
            return c
    return m


def _spmm_kernel(adj_ref, sup_ref, b_ref, out_ref):
    a16 = adj_ref[...].astype(jnp.bfloat16)
    s16 = sup_ref[...].astype(jnp.bfloat16)
    acc = jnp.dot(a16, s16, preferred_element_type=jnp.float32)
    out_ref[...] = _leaky(acc + b_ref[...])


def _spmm(adj, sup, b, bm):
    m, k = adj.shape
    n = sup.shape[1]
    b2 = b.reshape(1, n)
    return pl.pallas_call(
        _spmm_kernel,
        grid=(m // bm,),
        in_specs=[
            pl.BlockSpec((bm, k), lambda i: (i, 0)),
            pl.BlockSpec((k, n), lambda i: (0, 0)),
            pl.BlockSpec((1, n), lambda i: (0, 0)),
        ],
        out_specs=pl.BlockSpec((bm, n), lambda i: (i, 0)),
        out_shape=jax.ShapeDtypeStruct((m, n), jnp.float32),
        compiler_params=pltpu.CompilerParams(
            dimension_semantics=("parallel",)
        ),
    )(adj, sup, b2)


def _linN_kernel(*refs, nx, act):
    x_refs = refs[:nx]
    w_ref, b_ref, out_ref = refs[nx], refs[nx + 1], refs[nx + 2]
    x = jnp.concatenate([r[...] for r in x_refs], axis=1) if nx > 1 else x_refs[0][...]
    y = jnp.dot(x, w_ref[...], preferred_element_type=jnp.float32) + b_ref[...]
    out_ref[...] = act(y)


def _fused_linear(xs, w, b, act, bm):
    m = xs[0].shape[0]
    fi, n = w.shape
    b2 = b.reshape(1, n)
    nx = len(xs)
    in_specs = [pl.BlockSpec((bm, x.shape[1]), lambda i: (i, 0)) for x in xs]
    in_specs += [
        pl.BlockSpec((fi, n), lambda i: (0, 0)),
        pl.BlockSpec((1, n), lambda i: (0, 0)),
    ]
    return pl.pallas_call(
        functools.partial(_linN_kernel, nx=nx, act=act),
        grid=(m // bm,),
        in_specs=in_specs,
        out_specs=pl.BlockSpec((bm, n), lambda i: (i, 0)),
        out_shape=jax.ShapeDtypeStruct((m, n), jnp.float32),
        compiler_params=pltpu.CompilerParams(
            dimension_semantics=("parallel",)
        ),
    )(*xs, w, b2)


def _pair_kernel(a_ref, b_ref, w1_ref, w2_ref, out_ref):
    ya = jnp.dot(a_ref[...], w1_ref[...], preferred_element_type=jnp.float32)
    yb = jnp.dot(b_ref[...], w2_ref[...], preferred_element_type=jnp.float32)
    out_ref[...] = jnp.concatenate([ya, yb], axis=1)


def _pair_support(a, b, w1, w2, bm):
    m, d = a.shape
    n1 = w1.shape[1]
    n2 = w2.shape[1]
    return pl.pallas_call(
        _pair_kernel,
        grid=(m // bm,),
        in_specs=[
            pl.BlockSpec((bm, d), lambda i: (i, 0)),
            pl.BlockSpec((bm, b.shape[1]), lambda i: (i, 0)),
            pl.BlockSpec((d, n1), lambda i: (0, 0)),
            pl.BlockSpec((b.shape[1], n2), lambda i: (0, 0)),
        ],
        out_specs=pl.BlockSpec((bm, n1 + n2), lambda i: (i, 0)),
        out_shape=jax.ShapeDtypeStruct((m, n1 + n2), jnp.float32),
        compiler_params=pltpu.CompilerParams(
            dimension_semantics=("parallel",)
        ),
    )(a, b, w1, w2)


def _alpha_kernel(h0_ref, h1_ref, w1_ref, b1_ref, w2t_ref, hu_ref, alpha_ref):
    m0 = jnp.mean(h0_ref[...], axis=0, keepdims=True)
    m1 = jnp.mean(h1_ref[...], axis=0, keepdims=True)
    hu = jnp.concatenate([m0, m1], axis=0)
    hu_ref[...] = hu
    z = jnp.maximum(
        jnp.dot(hu, w1_ref[...], preferred_element_type=jnp.float32) + b1_ref[...],
        0.0,
    )
    logits = jnp.sum(z * w2t_ref[...], axis=1, keepdims=True)
    mx = jnp.max(logits, axis=0, keepdims=True)
    e = jnp.exp(logits - mx)
    a = e / jnp.sum(e, axis=0, keepdims=True)
    alpha_ref[...] = jnp.broadcast_to(a, alpha_ref.shape)


def _mean_alpha(h0, h1, w1, b1, w2):
    m, d = h0.shape
    b1r = b1.reshape(1, d)
    w2t = w2.reshape(1, d)
    return pl.pallas_call(
        _alpha_kernel,
        grid=(1,),
        in_specs=[
            pl.BlockSpec((m, d), lambda i: (0, 0)),
            pl.BlockSpec((m, d), lambda i: (0, 0)),
            pl.BlockSpec((d, d), lambda i: (0, 0)),
            pl.BlockSpec((1, d), lambda i: (0, 0)),
            pl.BlockSpec((1, d), lambda i: (0, 0)),
        ],
        out_specs=[
            pl.BlockSpec((2, d), lambda i: (0, 0)),
            pl.BlockSpec((2, d), lambda i: (0, 0)),
        ],
        out_shape=[
            jax.ShapeDtypeStruct((2, d), jnp.float32),
            jax.ShapeDtypeStruct((2, d), jnp.float32),
        ],
        compiler_params=pltpu.CompilerParams(
            dimension_semantics=("parallel",)
        ),
    )(h0, h1, w1, b1r, w2t)


def _combine_kernel(h0_ref, h1_ref, a_ref, out_ref):
    out_ref[...] = 0.5 * (
        a_ref[0:1, :] * h0_ref[...] + a_ref[1:2, :] * h1_ref[...]
    )


def _combine(h0, h1, alpha_bc, bm):
    m, d = h0.shape
    return pl.pallas_call(
        _combine_kernel,
        grid=(m // bm,),
        in_specs=[
            pl.BlockSpec((bm, d), lambda i: (i, 0)),
            pl.BlockSpec((bm, d), lambda i: (i, 0)),
            pl.BlockSpec((2, d), lambda i: (0, 0)),
        ],
        out_specs=pl.BlockSpec((bm, d), lambda i: (i, 0)),
        out_shape=jax.ShapeDtypeStruct((m, d), jnp.float32),
        compiler_params=pltpu.CompilerParams(
            dimension_semantics=("parallel",)
        ),
    )(h0, h1, alpha_bc)


def _relu(x):
    return jnp.maximum(x, 0.0)


def kernel(ufea, vfea, UV_adj, VU_adj, adj, params):
    p = params
    n_u = ufea.shape[0]
    n_i = vfea.shape[0]

    bm_spmm_u = _pick_bm(n_u, [400, 200, 80, 40, 16, 8])
    bm_spmm_i = _pick_bm(n_i, [400, 200, 80, 40, 16, 8])
    bm_lin_u = _pick_bm(n_u, [2000, 1000, 400, 80, 16, 8])
    bm_lin_i = _pick_bm(n_i, [2000, 1000, 400, 80, 16, 8])

    sup_u = _fused_linear([vfea], p['W_gc1_0'], jnp.zeros_like(p['b_gc1_0']),
                          lambda x: x, bm_lin_i)
    sup_i = _fused_linear([ufea], p['W_gc2_0'], jnp.zeros_like(p['b_gc2_0']),
                          lambda x: x, bm_lin_u)
    User_n = _spmm(UV_adj, sup_u, p['b_gc1_0'], bm_spmm_u)
    Item_n = _spmm(VU_adj, sup_i, p['b_gc2_0'], bm_spmm_i)

    User_h0 = _fused_linear([ufea, User_n], p['W_uu0'], p['b_uu0'], _relu, bm_lin_u)
    Item_h0 = _fused_linear([vfea, Item_n], p['W_iu0'], p['b_iu0'], _relu, bm_lin_i)

    sup_uv = _pair_support(Item_h0, Item_n, p['W_gc3_1'], p['W_gc1_1'], bm_lin_i)
    sup_vu = _pair_support(User_h0, User_n, p['W_gc4_1'], p['W_gc2_1'], bm_lin_u)
    b_uv = jnp.concatenate([p['b_gc3_1'], p['b_gc1_1']])
    b_vu = jnp.concatenate([p['b_gc4_1'], p['b_gc2_1']])
    Un = _spmm(UV_adj, sup_uv, b_uv, bm_spmm_u)
    In = _spmm(VU_adj, sup_vu, b_vu, bm_spmm_i)

    d = ufea.shape[1]
    User_n1 = Un[:, :d]
    User_n2 = Un[:, d:]
    Item_n1 = In[:, :d]
    Item_n2 = In[:, d:]

    learn_user = _fused_linear([User_h0, User_n2, User_n1], p['W_uu1'],
                               p['b_uu1'], _relu, bm_lin_u)
    learn_item = _fused_linear([Item_h0, Item_n2, Item_n1], p['W_iu1'],
                               p['b_iu1'], _relu, bm_lin_i)

    Hu, alpha_u_bc = _mean_alpha(User_h0, learn_user, p['W_mlp_ul'],
                                 p['b_mlp_ul'], p['W_mlp_ul1'])
    Hv, alpha_v_bc = _mean_alpha(Item_h0, learn_item, p['W_mlp_vl'],
                                 p['b_mlp_vl'], p['W_mlp_vl1'])

    h_u_final = _combine(User_h0, learn_user, alpha_u_bc, bm_lin_u)
    h_v_final = _combine(Item_h0, learn_item, alpha_v_bc, bm_lin_i)

    alpha_ul = alpha_u_bc[:, :1]
    alpha_vl = alpha_v_bc[:, :1]

    return (learn_user, learn_item, h_u_final, h_v_final,
            alpha_ul, alpha_vl, Hu, Hv)


# speedup vs baseline: 1.1331x; 1.1331x over previous
import functools

import jax
import jax.numpy as jnp
from jax.experimental import pallas as pl
from jax.experimental.pallas import tpu as pltpu

_ALPHA = 0.2


def _leaky(x):
    return jnp.where(x > 0, x, _ALPHA * x)


def _relu(x):
    return jnp.maximum(x, 0.0)


def _pick_bm(m, candidates):
    for c in candidates:
        if m % c == 0:
            return c
    return m


def _lin_kernel(x_ref, w_ref, out_ref):
    out_ref[...] = jnp.dot(x_ref[...], w_ref[...],
                           preferred_element_type=jnp.float32)


def _linear(x, w, bm):
    m, d = x.shape
    n = w.shape[1]
    return pl.pallas_call(
        _lin_kernel,
        grid=(m // bm,),
        in_specs=[
            pl.BlockSpec((bm, d), lambda i: (i, 0)),
            pl.BlockSpec((d, n), lambda i: (0, 0)),
        ],
        out_specs=pl.BlockSpec((bm, n), lambda i: (i, 0)),
        out_shape=jax.ShapeDtypeStruct((m, n), jnp.float32),
        compiler_params=pltpu.CompilerParams(
            dimension_semantics=("arbitrary",)
        ),
    )(x, w)


def _pass1_kernel(adj_ref, sup_ref, b_ref, x_ref, wh_ref, bh_ref,
                  wpa_ref, wpb_ref, h0_ref, pair_ref):
    acc = jnp.dot(adj_ref[...], sup_ref[...],
                  preferred_element_type=jnp.float32)
    n = _leaky(acc + b_ref[...])
    d = n.shape[1]
    wh = wh_ref[...]
    h0 = _relu(jnp.dot(x_ref[...], wh[:d, :],
                       preferred_element_type=jnp.float32)
               + jnp.dot(n, wh[d:, :], preferred_element_type=jnp.float32)
               + bh_ref[...])
    h0_ref[...] = h0
    pa = jnp.dot(h0, wpa_ref[...], preferred_element_type=jnp.float32)
    pb = jnp.dot(n, wpb_ref[...], preferred_element_type=jnp.float32)
    pair_ref[...] = jnp.concatenate([pa, pb], axis=1)


def _pass1(adj, sup, b, x, wh, bh, wpa, wpb, bm):
    m, k = adj.shape
    d = sup.shape[1]
    return pl.pallas_call(
        _pass1_kernel,
        grid=(m // bm,),
        in_specs=[
            pl.BlockSpec((bm, k), lambda i: (i, 0)),
            pl.BlockSpec((k, d), lambda i: (0, 0)),
            pl.BlockSpec((1, d), lambda i: (0, 0)),
            pl.BlockSpec((bm, d), lambda i: (i, 0)),
            pl.BlockSpec((2 * d, d), lambda i: (0, 0)),
            pl.BlockSpec((1, d), lambda i: (0, 0)),
            pl.BlockSpec((d, d), lambda i: (0, 0)),
            pl.BlockSpec((d, d), lambda i: (0, 0)),
        ],
        out_specs=[
            pl.BlockSpec((bm, d), lambda i: (i, 0)),
            pl.BlockSpec((bm, 2 * d), lambda i: (i, 0)),
        ],
        out_shape=[
            jax.ShapeDtypeStruct((m, d), jnp.float32),
            jax.ShapeDtypeStruct((m, 2 * d), jnp.float32),
        ],
        compiler_params=pltpu.CompilerParams(
            dimension_semantics=("arbitrary",)
        ),
    )(adj, sup, b.reshape(1, d), x, wh, bh.reshape(1, d), wpa, wpb)


def _pass2_kernel(adj_ref, sup_ref, b_ref, h0_ref, w_ref, bl_ref,
                  learn_ref, hmean_ref, *, inv_m):
    acc = jnp.dot(adj_ref[...], sup_ref[...],
                  preferred_element_type=jnp.float32)
    un = _leaky(acc + b_ref[...])
    d = h0_ref.shape[1]
    n1 = un[:, :d]
    n2 = un[:, d:]
    w = w_ref[...]
    h0 = h0_ref[...]
    learn = _relu(
        jnp.dot(h0, w[:d, :], preferred_element_type=jnp.float32)
        + jnp.dot(n2, w[d:2 * d, :], preferred_element_type=jnp.float32)
        + jnp.dot(n1, w[2 * d:, :], preferred_element_type=jnp.float32)
        + bl_ref[...])
    learn_ref[...] = learn
    part = jnp.concatenate([
        jnp.sum(h0, axis=0, keepdims=True) * inv_m,
        jnp.sum(learn, axis=0, keepdims=True) * inv_m,
    ], axis=0)

    @pl.when(pl.program_id(0) == 0)
    def _init():
        hmean_ref[...] = part

    @pl.when(pl.program_id(0) != 0)
    def _acc():
        hmean_ref[...] += part


def _pass2(adj, sup256, bcat, h0, w, bl, bm):
    m, k = adj.shape
    d = h0.shape[1]
    return pl.pallas_call(
        functools.partial(_pass2_kernel, inv_m=1.0 / m),
        grid=(m // bm,),
        in_specs=[
            pl.BlockSpec((bm, k), lambda i: (i, 0)),
            pl.BlockSpec((k, 2 * d), lambda i: (0, 0)),
            pl.BlockSpec((1, 2 * d), lambda i: (0, 0)),
            pl.BlockSpec((bm, d), lambda i: (i, 0)),
            pl.BlockSpec((3 * d, d), lambda i: (0, 0)),
            pl.BlockSpec((1, d), lambda i: (0, 0)),
        ],
        out_specs=[
            pl.BlockSpec((bm, d), lambda i: (i, 0)),
            pl.BlockSpec((2, d), lambda i: (0, 0)),
        ],
        out_shape=[
            jax.ShapeDtypeStruct((m, d), jnp.float32),
            jax.ShapeDtypeStruct((2, d), jnp.float32),
        ],
        compiler_params=pltpu.CompilerParams(
            dimension_semantics=("arbitrary",)
        ),
    )(adj, sup256, bcat.reshape(1, 2 * d), h0, w, bl.reshape(1, d))


def _alpha_kernel(hu_ref, hv_ref, wu_ref, bu_ref, w2u_ref,
                  wv_ref, bv_ref, w2v_ref, au_ref, av_ref):
    def one(h_ref, w_ref, b_ref, w2_ref, a_ref):
        z = _relu(jnp.dot(h_ref[...], w_ref[...],
                          preferred_element_type=jnp.float32) + b_ref[...])
        logits = jnp.sum(z * w2_ref[...], axis=1, keepdims=True)
        mx = jnp.max(logits, axis=0, keepdims=True)
        e = jnp.exp(logits - mx)
        a = e / jnp.sum(e, axis=0, keepdims=True)
        a_ref[...] = jnp.broadcast_to(a, a_ref.shape)

    one(hu_ref, wu_ref, bu_ref, w2u_ref, au_ref)
    one(hv_ref, wv_ref, bv_ref, w2v_ref, av_ref)


def _alphas(hu, hv, wu, bu, w2u, wv, bv, w2v):
    d = hu.shape[1]
    full = lambda i: (0, 0)
    return pl.pallas_call(
        _alpha_kernel,
        grid=(1,),
        in_specs=[
            pl.BlockSpec((2, d), full),
            pl.BlockSpec((2, d), full),
            pl.BlockSpec((d, d), full),
            pl.BlockSpec((1, d), full),
            pl.BlockSpec((1, d), full),
            pl.BlockSpec((d, d), full),
            pl.BlockSpec((1, d), full),
            pl.BlockSpec((1, d), full),
        ],
        out_specs=[
            pl.BlockSpec((2, d), full),
            pl.BlockSpec((2, d), full),
        ],
        out_shape=[
            jax.ShapeDtypeStruct((2, d), jnp.float32),
            jax.ShapeDtypeStruct((2, d), jnp.float32),
        ],
        compiler_params=pltpu.CompilerParams(
            dimension_semantics=("arbitrary",)
        ),
    )(hu, hv, wu, bu.reshape(1, d), w2u.reshape(1, d),
      wv, bv.reshape(1, d), w2v.reshape(1, d))


def _combine_kernel(h0_ref, h1_ref, a_ref, out_ref):
    out_ref[...] = 0.5 * (
        a_ref[0:1, :] * h0_ref[...] + a_ref[1:2, :] * h1_ref[...]
    )


def _combine(h0, h1, alpha_bc, bm):
    m, d = h0.shape
    return pl.pallas_call(
        _combine_kernel,
        grid=(m // bm,),
        in_specs=[
            pl.BlockSpec((bm, d), lambda i: (i, 0)),
            pl.BlockSpec((bm, d), lambda i: (i, 0)),
            pl.BlockSpec((2, d), lambda i: (0, 0)),
        ],
        out_specs=pl.BlockSpec((bm, d), lambda i: (i, 0)),
        out_shape=jax.ShapeDtypeStruct((m, d), jnp.float32),
        compiler_params=pltpu.CompilerParams(
            dimension_semantics=("arbitrary",)
        ),
    )(h0, h1, alpha_bc)


def kernel(ufea, vfea, UV_adj, VU_adj, adj, params):
    p = params
    n_u = ufea.shape[0]
    n_i = vfea.shape[0]

    bm_u = _pick_bm(n_u, [400, 200, 80, 40, 16, 8])
    bm_i = _pick_bm(n_i, [400, 200, 80, 40, 16, 8])
    bm_lin_u = _pick_bm(n_u, [2000, 1000, 400, 80, 16, 8])
    bm_lin_i = _pick_bm(n_i, [2000, 1000, 400, 80, 16, 8])

    sup_u = _linear(vfea, p['W_gc1_0'], bm_lin_i)
    sup_i = _linear(ufea, p['W_gc2_0'], bm_lin_u)

    User_h0, sup_vu = _pass1(UV_adj, sup_u, p['b_gc1_0'], ufea,
                             p['W_uu0'], p['b_uu0'],
                             p['W_gc4_1'], p['W_gc2_1'], bm_u)
    Item_h0, sup_uv = _pass1(VU_adj, sup_i, p['b_gc2_0'], vfea,
                             p['W_iu0'], p['b_iu0'],
                             p['W_gc3_1'], p['W_gc1_1'], bm_i)

    b_uv = jnp.concatenate([p['b_gc3_1'], p['b_gc1_1']])
    b_vu = jnp.concatenate([p['b_gc4_1'], p['b_gc2_1']])
    learn_user, Hu = _pass2(UV_adj, sup_uv, b_uv, User_h0,
                            p['W_uu1'], p['b_uu1'], bm_u)
    learn_item, Hv = _pass2(VU_adj, sup_vu, b_vu, Item_h0,
                            p['W_iu1'], p['b_iu1'], bm_i)

    alpha_u_bc, alpha_v_bc = _alphas(
        Hu, Hv,
        p['W_mlp_ul'], p['b_mlp_ul'], p['W_mlp_ul1'],
        p['W_mlp_vl'], p['b_mlp_vl'], p['W_mlp_vl1'])

    h_u_final = _combine(User_h0, learn_user, alpha_u_bc, bm_lin_u)
    h_v_final = _combine(Item_h0, learn_item, alpha_v_bc, bm_lin_i)

    alpha_ul = alpha_u_bc[:, :1]
    alpha_vl = alpha_v_bc[:, :1]

    return (learn_user, learn_item, h_u_final, h_v_final,
            alpha_ul, alpha_vl, Hu, Hv)


# speedup vs baseline: 1.1334x; 1.0003x over previous
import functools

import jax
import jax.numpy as jnp
from jax.experimental import pallas as pl
from jax.experimental.pallas import tpu as pltpu

_ALPHA = 0.2


def _leaky(x):
    return jnp.where(x > 0, x, _ALPHA * x)


def _relu(x):
    return jnp.maximum(x, 0.0)


def _pick_bm(m, candidates):
    for c in candidates:
        if m % c == 0:
            return c
    return m


def _lin_kernel(x_ref, w_ref, out_ref):
    out_ref[...] = jnp.dot(x_ref[...], w_ref[...],
                           preferred_element_type=jnp.float32)


def _linear(x, w, bm):
    m, d = x.shape
    n = w.shape[1]
    return pl.pallas_call(
        _lin_kernel,
        grid=(m // bm,),
        in_specs=[
            pl.BlockSpec((bm, d), lambda i: (i, 0)),
            pl.BlockSpec((d, n), lambda i: (0, 0)),
        ],
        out_specs=pl.BlockSpec((bm, n), lambda i: (i, 0)),
        out_shape=jax.ShapeDtypeStruct((m, n), jnp.float32),
        compiler_params=pltpu.CompilerParams(
            dimension_semantics=("arbitrary",)
        ),
    )(x, w)


def _pass1_kernel(adj_ref, sup_ref, b_ref, x_ref, wh_ref, bh_ref,
                  wpa_ref, wpb_ref, h0_ref, pair_ref):
    acc = jnp.dot(adj_ref[...].astype(jnp.bfloat16),
                  sup_ref[...].astype(jnp.bfloat16),
                  preferred_element_type=jnp.float32)
    n = _leaky(acc + b_ref[...])
    d = n.shape[1]
    wh = wh_ref[...]
    h0 = _relu(jnp.dot(x_ref[...], wh[:d, :],
                       preferred_element_type=jnp.float32)
               + jnp.dot(n, wh[d:, :], preferred_element_type=jnp.float32)
               + bh_ref[...])
    h0_ref[...] = h0
    pa = jnp.dot(h0, wpa_ref[...], preferred_element_type=jnp.float32)
    pb = jnp.dot(n, wpb_ref[...], preferred_element_type=jnp.float32)
    pair_ref[...] = jnp.concatenate([pa, pb], axis=1)


def _pass1(adj, sup, b, x, wh, bh, wpa, wpb, bm):
    m, k = adj.shape
    d = sup.shape[1]
    return pl.pallas_call(
        _pass1_kernel,
        grid=(m // bm,),
        in_specs=[
            pl.BlockSpec((bm, k), lambda i: (i, 0)),
            pl.BlockSpec((k, d), lambda i: (0, 0)),
            pl.BlockSpec((1, d), lambda i: (0, 0)),
            pl.BlockSpec((bm, d), lambda i: (i, 0)),
            pl.BlockSpec((2 * d, d), lambda i: (0, 0)),
            pl.BlockSpec((1, d), lambda i: (0, 0)),
            pl.BlockSpec((d, d), lambda i: (0, 0)),
            pl.BlockSpec((d, d), lambda i: (0, 0)),
        ],
        out_specs=[
            pl.BlockSpec((bm, d), lambda i: (i, 0)),
            pl.BlockSpec((bm, 2 * d), lambda i: (i, 0)),
        ],
        out_shape=[
            jax.ShapeDtypeStruct((m, d), jnp.float32),
            jax.ShapeDtypeStruct((m, 2 * d), jnp.float32),
        ],
        compiler_params=pltpu.CompilerParams(
            dimension_semantics=("arbitrary",)
        ),
    )(adj, sup, b.reshape(1, d), x, wh, bh.reshape(1, d), wpa, wpb)


def _pass2_kernel(adj_ref, sup_ref, b_ref, h0_ref, w_ref, bl_ref,
                  learn_ref, hmean_ref, *, inv_m):
    acc = jnp.dot(adj_ref[...].astype(jnp.bfloat16),
                  sup_ref[...].astype(jnp.bfloat16),
                  preferred_element_type=jnp.float32)
    un = _leaky(acc + b_ref[...])
    d = h0_ref.shape[1]
    n1 = un[:, :d]
    n2 = un[:, d:]
    w = w_ref[...]
    h0 = h0_ref[...]
    learn = _relu(
        jnp.dot(h0, w[:d, :], preferred_element_type=jnp.float32)
        + jnp.dot(n2, w[d:2 * d, :], preferred_element_type=jnp.float32)
        + jnp.dot(n1, w[2 * d:, :], preferred_element_type=jnp.float32)
        + bl_ref[...])
    learn_ref[...] = learn
    part = jnp.concatenate([
        jnp.sum(h0, axis=0, keepdims=True) * inv_m,
        jnp.sum(learn, axis=0, keepdims=True) * inv_m,
    ], axis=0)

    @pl.when(pl.program_id(0) == 0)
    def _init():
        hmean_ref[...] = part

    @pl.when(pl.program_id(0) != 0)
    def _acc():
        hmean_ref[...] += part


def _pass2(adj, sup256, bcat, h0, w, bl, bm):
    m, k = adj.shape
    d = h0.shape[1]
    return pl.pallas_call(
        functools.partial(_pass2_kernel, inv_m=1.0 / m),
        grid=(m // bm,),
        in_specs=[
            pl.BlockSpec((bm, k), lambda i: (i, 0)),
            pl.BlockSpec((k, 2 * d), lambda i: (0, 0)),
            pl.BlockSpec((1, 2 * d), lambda i: (0, 0)),
            pl.BlockSpec((bm, d), lambda i: (i, 0)),
            pl.BlockSpec((3 * d, d), lambda i: (0, 0)),
            pl.BlockSpec((1, d), lambda i: (0, 0)),
        ],
        out_specs=[
            pl.BlockSpec((bm, d), lambda i: (i, 0)),
            pl.BlockSpec((2, d), lambda i: (0, 0)),
        ],
        out_shape=[
            jax.ShapeDtypeStruct((m, d), jnp.float32),
            jax.ShapeDtypeStruct((2, d), jnp.float32),
        ],
        compiler_params=pltpu.CompilerParams(
            dimension_semantics=("arbitrary",)
        ),
    )(adj, sup256, bcat.reshape(1, 2 * d), h0, w, bl.reshape(1, d))


def _alpha_kernel(hu_ref, hv_ref, wu_ref, bu_ref, w2u_ref,
                  wv_ref, bv_ref, w2v_ref, au_ref, av_ref):
    def one(h_ref, w_ref, b_ref, w2_ref, a_ref):
        z = _relu(jnp.dot(h_ref[...], w_ref[...],
                          preferred_element_type=jnp.float32) + b_ref[...])
        logits = jnp.sum(z * w2_ref[...], axis=1, keepdims=True)
        mx = jnp.max(logits, axis=0, keepdims=True)
        e = jnp.exp(logits - mx)
        a = e / jnp.sum(e, axis=0, keepdims=True)
        a_ref[...] = jnp.broadcast_to(a, a_ref.shape)

    one(hu_ref, wu_ref, bu_ref, w2u_ref, au_ref)
    one(hv_ref, wv_ref, bv_ref, w2v_ref, av_ref)


def _alphas(hu, hv, wu, bu, w2u, wv, bv, w2v):
    d = hu.shape[1]
    full = lambda i: (0, 0)
    return pl.pallas_call(
        _alpha_kernel,
        grid=(1,),
        in_specs=[
            pl.BlockSpec((2, d), full),
            pl.BlockSpec((2, d), full),
            pl.BlockSpec((d, d), full),
            pl.BlockSpec((1, d), full),
            pl.BlockSpec((1, d), full),
            pl.BlockSpec((d, d), full),
            pl.BlockSpec((1, d), full),
            pl.BlockSpec((1, d), full),
        ],
        out_specs=[
            pl.BlockSpec((2, d), full),
            pl.BlockSpec((2, d), full),
        ],
        out_shape=[
            jax.ShapeDtypeStruct((2, d), jnp.float32),
            jax.ShapeDtypeStruct((2, d), jnp.float32),
        ],
        compiler_params=pltpu.CompilerParams(
            dimension_semantics=("arbitrary",)
        ),
    )(hu, hv, wu, bu.reshape(1, d), w2u.reshape(1, d),
      wv, bv.reshape(1, d), w2v.reshape(1, d))


def _combine_kernel(h0_ref, h1_ref, a_ref, out_ref):
    out_ref[...] = 0.5 * (
        a_ref[0:1, :] * h0_ref[...] + a_ref[1:2, :] * h1_ref[...]
    )


def _combine(h0, h1, alpha_bc, bm):
    m, d = h0.shape
    return pl.pallas_call(
        _combine_kernel,
        grid=(m // bm,),
        in_specs=[
            pl.BlockSpec((bm, d), lambda i: (i, 0)),
            pl.BlockSpec((bm, d), lambda i: (i, 0)),
            pl.BlockSpec((2, d), lambda i: (0, 0)),
        ],
        out_specs=pl.BlockSpec((bm, d), lambda i: (i, 0)),
        out_shape=jax.ShapeDtypeStruct((m, d), jnp.float32),
        compiler_params=pltpu.CompilerParams(
            dimension_semantics=("arbitrary",)
        ),
    )(h0, h1, alpha_bc)


def kernel(ufea, vfea, UV_adj, VU_adj, adj, params):
    p = params
    n_u = ufea.shape[0]
    n_i = vfea.shape[0]

    bm_u = _pick_bm(n_u, [400, 200, 80, 40, 16, 8])
    bm_i = _pick_bm(n_i, [400, 200, 80, 40, 16, 8])
    bm_lin_u = _pick_bm(n_u, [2000, 1000, 400, 80, 16, 8])
    bm_lin_i = _pick_bm(n_i, [2000, 1000, 400, 80, 16, 8])

    sup_u = _linear(vfea, p['W_gc1_0'], bm_lin_i)
    sup_i = _linear(ufea, p['W_gc2_0'], bm_lin_u)

    User_h0, sup_vu = _pass1(UV_adj, sup_u, p['b_gc1_0'], ufea,
                             p['W_uu0'], p['b_uu0'],
                             p['W_gc4_1'], p['W_gc2_1'], bm_u)
    Item_h0, sup_uv = _pass1(VU_adj, sup_i, p['b_gc2_0'], vfea,
                             p['W_iu0'], p['b_iu0'],
                             p['W_gc3_1'], p['W_gc1_1'], bm_i)

    b_uv = jnp.concatenate([p['b_gc3_1'], p['b_gc1_1']])
    b_vu = jnp.concatenate([p['b_gc4_1'], p['b_gc2_1']])
    learn_user, Hu = _pass2(UV_adj, sup_uv, b_uv, User_h0,
                            p['W_uu1'], p['b_uu1'], bm_u)
    learn_item, Hv = _pass2(VU_adj, sup_vu, b_vu, Item_h0,
                            p['W_iu1'], p['b_iu1'], bm_i)

    alpha_u_bc, alpha_v_bc = _alphas(
        Hu, Hv,
        p['W_mlp_ul'], p['b_mlp_ul'], p['W_mlp_ul1'],
        p['W_mlp_vl'], p['b_mlp_vl'], p['W_mlp_vl1'])

    h_u_final = _combine(User_h0, learn_user, alpha_u_bc, bm_lin_u)
    h_v_final = _combine(Item_h0, learn_item, alpha_v_bc, bm_lin_i)

    alpha_ul = alpha_u_bc[:, :1]
    alpha_vl = alpha_v_bc[:, :1]

    return (learn_user, learn_item, h_u_final, h_v_final,
            alpha_ul, alpha_vl, Hu, Hv)
